# Initial kernel scaffold; baseline (speedup 1.0000x reference)
import functools
import jax
import jax.numpy as jnp
from jax import lax
from jax.experimental import pallas as pl
from jax.experimental.pallas import tpu as pltpu
from jax.experimental.pallas import tpu_sc as plsc

NUM_EDGES = 320000
HIDDEN = 128
NC = 2
NS = 16
NW = NC * NS
PER_W = NUM_EDGES // NW
BLK = 200
NBLK = PER_W // BLK


def _body(src_hbm, dst_hbm, typ_hbm, z_hbm, rel_hbm, out_hbm,
          sidx, didx, tidx, srows, drows, rrows, oblk, sem):
    wid = lax.axis_index("s") * NC + lax.axis_index("c")
    wbase = wid * PER_W

    def block(b, _):
        base = wbase + b * BLK
        pltpu.sync_copy(src_hbm.at[pl.ds(base, BLK)], sidx)
        pltpu.sync_copy(dst_hbm.at[pl.ds(base, BLK)], didx)
        pltpu.sync_copy(typ_hbm.at[pl.ds(base, BLK)], tidx)
        pltpu.async_copy(z_hbm.at[sidx], srows, sem).wait()
        pltpu.async_copy(z_hbm.at[didx], drows, sem).wait()
        pltpu.async_copy(rel_hbm.at[tidx], rrows, sem).wait()

        def edge(e, _):
            acc = jnp.zeros((16,), jnp.float32)
            for k in range(HIDDEN // 16):
                sl = pl.ds(k * 16, 16)
                acc = acc + srows[e, sl] * rrows[e, sl] * drows[e, sl]
            oblk[e] = jnp.sum(acc)
            return 0

        lax.fori_loop(0, BLK, edge, 0)
        pltpu.sync_copy(oblk, out_hbm.at[pl.ds(base, BLK)])
        return 0

    lax.fori_loop(0, NBLK, block, 0)


@jax.jit
def _run(src, dst, typ, z, rel_emb):
    mesh = plsc.VectorSubcoreMesh(core_axis_name="c", subcore_axis_name="s",
                                  num_cores=NC, num_subcores=NS)
    kern = pl.kernel(
        _body,
        out_type=jax.ShapeDtypeStruct((NUM_EDGES,), jnp.float32),
        mesh=mesh,
        scratch_types=[
            pltpu.VMEM((BLK,), jnp.int32),
            pltpu.VMEM((BLK,), jnp.int32),
            pltpu.VMEM((BLK,), jnp.int32),
            pltpu.VMEM((BLK, HIDDEN), jnp.float32),
            pltpu.VMEM((BLK, HIDDEN), jnp.float32),
            pltpu.VMEM((BLK, HIDDEN), jnp.float32),
            pltpu.VMEM((BLK,), jnp.float32),
            pltpu.SemaphoreType.DMA,
        ],
    )
    return kern(src, dst, typ, z, rel_emb)


def kernel(z, edge_index, edge_type, rel_emb):
    src = edge_index[0].astype(jnp.int32)
    dst = edge_index[1].astype(jnp.int32)
    typ = edge_type.astype(jnp.int32)
    return _run(src, dst, typ, z, rel_emb)


# speedup vs baseline: 2.5534x; 2.5534x over previous
import functools
import jax
import jax.numpy as jnp
from jax import lax
from jax.experimental import pallas as pl
from jax.experimental.pallas import tpu as pltpu
from jax.experimental.pallas import tpu_sc as plsc

NUM_EDGES = 320000
HIDDEN = 128
NC = 2
NS = 16
NW = NC * NS
PER_W = NUM_EDGES // NW
BLK = 80
NBLK = PER_W // BLK


def _body(src_hbm, dst_hbm, typ_hbm, z_hbm, rel_hbm, out_hbm,
          sidx, didx, tidx, srows, drows, rrows, oblk, sem):
    wid = lax.axis_index("s") * NC + lax.axis_index("c")
    wbase = wid * PER_W

    def block(b, _):
        base = wbase + b * BLK
        pltpu.sync_copy(src_hbm.at[pl.ds(base, BLK)], sidx)
        pltpu.sync_copy(dst_hbm.at[pl.ds(base, BLK)], didx)
        pltpu.sync_copy(typ_hbm.at[pl.ds(base, BLK)], tidx)
        pltpu.async_copy(z_hbm.at[sidx], srows, sem).wait()
        pltpu.async_copy(z_hbm.at[didx], drows, sem).wait()
        pltpu.async_copy(rel_hbm.at[tidx], rrows, sem).wait()

        lanes = lax.iota(jnp.int32, 16)

        def group(g, _):
            res = jnp.zeros((16,), jnp.float32)
            for j in range(16):
                e = g * 16 + j
                acc = jnp.zeros((16,), jnp.float32)
                for k in range(HIDDEN // 16):
                    sl = pl.ds(k * 16, 16)
                    acc = acc + srows[e, sl] * rrows[e, sl] * drows[e, sl]
                res = jnp.where(lanes == j, jnp.sum(acc), res)
            oblk[pl.ds(g * 16, 16)] = res
            return 0

        lax.fori_loop(0, BLK // 16, group, 0)
        pltpu.sync_copy(oblk, out_hbm.at[pl.ds(base, BLK)])
        return 0

    lax.fori_loop(0, NBLK, block, 0)


@jax.jit
def _run(src, dst, typ, z, rel_emb):
    mesh = plsc.VectorSubcoreMesh(core_axis_name="c", subcore_axis_name="s",
                                  num_cores=NC, num_subcores=NS)
    kern = pl.kernel(
        _body,
        out_type=jax.ShapeDtypeStruct((NUM_EDGES,), jnp.float32),
        mesh=mesh,
        compiler_params=pltpu.CompilerParams(needs_layout_passes=False),
        scratch_types=[
            pltpu.VMEM((BLK,), jnp.int32),
            pltpu.VMEM((BLK,), jnp.int32),
            pltpu.VMEM((BLK,), jnp.int32),
            pltpu.VMEM((BLK, HIDDEN), jnp.float32),
            pltpu.VMEM((BLK, HIDDEN), jnp.float32),
            pltpu.VMEM((BLK, HIDDEN), jnp.float32),
            pltpu.VMEM((BLK,), jnp.float32),
            pltpu.SemaphoreType.DMA,
        ],
    )
    return kern(src, dst, typ, z, rel_emb)


def kernel(z, edge_index, edge_type, rel_emb):
    src = edge_index[0].astype(jnp.int32)
    dst = edge_index[1].astype(jnp.int32)
    typ = edge_type.astype(jnp.int32)
    return _run(src, dst, typ, z, rel_emb)


# speedup vs baseline: 3.2058x; 1.2555x over previous
import jax
import jax.numpy as jnp
from jax import lax
from jax.experimental import pallas as pl
from jax.experimental.pallas import tpu as pltpu
from jax.experimental.pallas import tpu_sc as plsc

NUM_EDGES = 320000
HIDDEN = 128
NCH = HIDDEN // 16
NC = 2
NS = 16
NW = NC * NS
PER_W = NUM_EDGES // NW
BLK = 80
NBLK = PER_W // BLK


def _body(src_hbm, dst_hbm, typ_hbm, z_hbm, rel_hbm, out_hbm,
          sidx, didx, tidx,
          srows0, drows0, rrows0, srows1, drows1, rrows1,
          obuf, sem0, sem1):
    wid = lax.axis_index("s") * NC + lax.axis_index("c")
    wbase = wid * PER_W

    bufs = ((srows0, drows0, rrows0, sem0), (srows1, drows1, rrows1, sem1))
    lanes = lax.iota(jnp.int32, 16)

    pltpu.sync_copy(src_hbm.at[pl.ds(wbase, PER_W)], sidx)
    pltpu.sync_copy(dst_hbm.at[pl.ds(wbase, PER_W)], didx)
    pltpu.sync_copy(typ_hbm.at[pl.ds(wbase, PER_W)], tidx)

    def issue(b, parity):
        sb, db, rb, sem = bufs[parity]
        sl = pl.ds(b * BLK, BLK)
        pltpu.async_copy(z_hbm.at[sidx.at[sl]], sb, sem)
        pltpu.async_copy(z_hbm.at[didx.at[sl]], db, sem)
        pltpu.async_copy(rel_hbm.at[tidx.at[sl]], rb, sem)

    def drain(b, parity):
        sb, db, rb, sem = bufs[parity]
        sl = pl.ds(b * BLK, BLK)
        pltpu.make_async_copy(z_hbm.at[sidx.at[sl]], sb, sem).wait()
        pltpu.make_async_copy(z_hbm.at[didx.at[sl]], db, sem).wait()
        pltpu.make_async_copy(rel_hbm.at[tidx.at[sl]], rb, sem).wait()

    def compute_blk(b, parity):
        sb, db, rb, _ = bufs[parity]

        def group(g, _):
            base = g * 16
            res = jnp.zeros((16,), jnp.float32)
            for j in range(16):
                e = base + j
                acc = jnp.zeros((16,), jnp.float32)
                for k in range(NCH):
                    sl = pl.ds(k * 16, 16)
                    acc = acc + sb[e, sl] * rb[e, sl] * db[e, sl]
                res = jnp.where(lanes == j, jnp.sum(acc), res)
            obuf[pl.ds(b * BLK + base, 16)] = res
            return 0

        lax.fori_loop(0, BLK // 16, group, 0)

    issue(0, 0)

    def pair(i, _):
        b0 = 2 * i
        b1 = 2 * i + 1
        drain(b0, 0)
        issue(b1, 1)
        compute_blk(b0, 0)
        drain(b1, 1)
        issue(b1 + 1, 0)
        compute_blk(b1, 1)
        return 0

    lax.fori_loop(0, NBLK // 2, pair, 0)

    drain(NBLK - 1, 0)
    compute_blk(NBLK - 1, 0)

    pltpu.sync_copy(obuf, out_hbm.at[pl.ds(wbase, PER_W)])


@jax.jit
def _run(src, dst, typ, z, rel_emb):
    mesh = plsc.VectorSubcoreMesh(core_axis_name="c", subcore_axis_name="s",
                                  num_cores=NC, num_subcores=NS)
    kern = pl.kernel(
        _body,
        out_type=jax.ShapeDtypeStruct((NUM_EDGES,), jnp.float32),
        mesh=mesh,
        compiler_params=pltpu.CompilerParams(needs_layout_passes=False),
        scratch_types=[
            pltpu.VMEM((PER_W,), jnp.int32),
            pltpu.VMEM((PER_W,), jnp.int32),
            pltpu.VMEM((PER_W,), jnp.int32),
            pltpu.VMEM((BLK, HIDDEN), jnp.float32),
            pltpu.VMEM((BLK, HIDDEN), jnp.float32),
            pltpu.VMEM((BLK, HIDDEN), jnp.float32),
            pltpu.VMEM((BLK, HIDDEN), jnp.float32),
            pltpu.VMEM((BLK, HIDDEN), jnp.float32),
            pltpu.VMEM((BLK, HIDDEN), jnp.float32),
            pltpu.VMEM((PER_W,), jnp.float32),
            pltpu.SemaphoreType.DMA,
            pltpu.SemaphoreType.DMA,
        ],
    )
    return kern(src, dst, typ, z, rel_emb)


def kernel(z, edge_index, edge_type, rel_emb):
    src = edge_index[0].astype(jnp.int32)
    dst = edge_index[1].astype(jnp.int32)
    typ = edge_type.astype(jnp.int32)
    return _run(src, dst, typ, z, rel_emb)


# speedup vs baseline: 7.9810x; 2.4896x over previous
import jax
import jax.numpy as jnp
from jax import lax
from jax.experimental import pallas as pl
from jax.experimental.pallas import tpu as pltpu
from jax.experimental.pallas import tpu_sc as plsc

NUM_EDGES = 320000
HIDDEN = 128
NCH = HIDDEN // 16
NC = 2
NS = 16
NW = NC * NS
PER_W = NUM_EDGES // NW
BLK = 80
NBLK = PER_W // BLK


def _body(src_hbm, dst_hbm, typ_hbm, z_hbm, rel_hbm, out_hbm,
          sidx, didx, tidx,
          srows0, drows0, rrows0, srows1, drows1, rrows1,
          obuf, sem0, sem1):
    wid = lax.axis_index("s") * NC + lax.axis_index("c")
    wbase = wid * PER_W

    bufs = ((srows0, drows0, rrows0, sem0), (srows1, drows1, rrows1, sem1))
    lanes = lax.iota(jnp.int32, 16)

    pltpu.sync_copy(src_hbm.at[pl.ds(wbase, PER_W)], sidx)
    pltpu.sync_copy(dst_hbm.at[pl.ds(wbase, PER_W)], didx)
    pltpu.sync_copy(typ_hbm.at[pl.ds(wbase, PER_W)], tidx)

    def issue(b, parity):
        sb, db, rb, sem = bufs[parity]
        sl = pl.ds(b * BLK, BLK)
        pltpu.async_copy(z_hbm.at[sidx.at[sl]], sb, sem)
        pltpu.async_copy(z_hbm.at[didx.at[sl]], db, sem)
        pltpu.async_copy(rel_hbm.at[tidx.at[sl]], rb, sem)

    def drain(b, parity):
        sb, db, rb, sem = bufs[parity]
        sl = pl.ds(b * BLK, BLK)
        pltpu.make_async_copy(z_hbm.at[sidx.at[sl]], sb, sem).wait()
        pltpu.make_async_copy(z_hbm.at[didx.at[sl]], db, sem).wait()
        pltpu.make_async_copy(rel_hbm.at[tidx.at[sl]], rb, sem).wait()

    last_lane = lanes == 15

    def compute_blk(b, parity):
        sb, db, rb, _ = bufs[parity]
        obase = b * BLK

        def edge(j, _):
            acc = jnp.zeros((16,), jnp.float32)
            for k in range(NCH):
                sl = pl.ds(k * 16, 16)
                acc = acc + sb[j, sl] * rb[j, sl] * db[j, sl]
            tot = plsc.cumsum(acc)
            idx = jnp.full((16,), obase + j, jnp.int32)
            plsc.store_scatter(obuf, [idx], tot, mask=last_lane)
            return 0

        lax.fori_loop(0, BLK, edge, 0)

    issue(0, 0)

    def pair(i, _):
        b0 = 2 * i
        b1 = 2 * i + 1
        drain(b0, 0)
        issue(b1, 1)
        compute_blk(b0, 0)
        drain(b1, 1)
        issue(b1 + 1, 0)
        compute_blk(b1, 1)
        return 0

    lax.fori_loop(0, NBLK // 2, pair, 0)

    drain(NBLK - 1, 0)
    compute_blk(NBLK - 1, 0)

    pltpu.sync_copy(obuf, out_hbm.at[pl.ds(wbase, PER_W)])


@jax.jit
def _run(src, dst, typ, z, rel_emb):
    mesh = plsc.VectorSubcoreMesh(core_axis_name="c", subcore_axis_name="s",
                                  num_cores=NC, num_subcores=NS)
    kern = pl.kernel(
        _body,
        out_type=jax.ShapeDtypeStruct((NUM_EDGES,), jnp.float32),
        mesh=mesh,
        compiler_params=pltpu.CompilerParams(needs_layout_passes=False),
        scratch_types=[
            pltpu.VMEM((PER_W,), jnp.int32),
            pltpu.VMEM((PER_W,), jnp.int32),
            pltpu.VMEM((PER_W,), jnp.int32),
            pltpu.VMEM((BLK, HIDDEN), jnp.float32),
            pltpu.VMEM((BLK, HIDDEN), jnp.float32),
            pltpu.VMEM((BLK, HIDDEN), jnp.float32),
            pltpu.VMEM((BLK, HIDDEN), jnp.float32),
            pltpu.VMEM((BLK, HIDDEN), jnp.float32),
            pltpu.VMEM((BLK, HIDDEN), jnp.float32),
            pltpu.VMEM((PER_W,), jnp.float32),
            pltpu.SemaphoreType.DMA,
            pltpu.SemaphoreType.DMA,
        ],
    )
    return kern(src, dst, typ, z, rel_emb)


def kernel(z, edge_index, edge_type, rel_emb):
    src = edge_index[0].astype(jnp.int32)
    dst = edge_index[1].astype(jnp.int32)
    typ = edge_type.astype(jnp.int32)
    return _run(src, dst, typ, z, rel_emb)
